# baseline (speedup 1.0000x reference)
import functools

import jax
import jax.numpy as jnp
from jax import lax
from jax.experimental import pallas as pl
from jax.experimental.pallas import tpu as pltpu
from jax.experimental.pallas import tpu_sc as plsc

CHUNK = 128
NBUF = 5
LOOK = 2


def _gather_rows(ids3, table):
    nw, n_chunks, _ = ids3.shape
    D = table.shape[1]
    N = nw * n_chunks * CHUNK
    rows_per_w = n_chunks * CHUNK
    info = plsc.get_sparse_core_info()
    nc = info.num_cores
    n_groups = n_chunks // NBUF

    mesh = plsc.VectorSubcoreMesh(core_axis_name="c", subcore_axis_name="s")

    @functools.partial(
        pl.kernel,
        mesh=mesh,
        out_type=jax.ShapeDtypeStruct((N, D), jnp.float32),
        scratch_types=[
            pltpu.VMEM((n_chunks, CHUNK), jnp.int32),
            *[pltpu.VMEM((CHUNK, D), jnp.float32) for _ in range(NBUF)],
            pltpu.SemaphoreType.DMA((NBUF,)),
            pltpu.SemaphoreType.DMA((NBUF,)),
        ],
    )
    def body(ids_hbm, table_hbm, out_hbm, idx_v, *rest):
        rows = rest[:NBUF]
        sem_g = rest[NBUF]
        sem_o = rest[NBUF + 1]
        wid = lax.axis_index("s") * nc + lax.axis_index("c")
        base_w = wid * rows_per_w

        pltpu.sync_copy(ids_hbm.at[wid], idx_v)

        def gather(j, b):
            pltpu.async_copy(table_hbm.at[idx_v.at[j]], rows[b], sem_g.at[b])

        def gather_wait(b):
            pltpu.make_async_copy(
                table_hbm.at[idx_v.at[0]], rows[b], sem_g.at[b]).wait()

        def scatter(j, b):
            pltpu.async_copy(
                rows[b], out_hbm.at[pl.ds(base_w + j * CHUNK, CHUNK)],
                sem_o.at[b])

        def scatter_wait(b):
            pltpu.make_async_copy(
                rows[b], out_hbm.at[pl.ds(base_w, CHUNK)], sem_o.at[b]).wait()

        for b in range(LOOK):
            gather(b, b)

        for b in range(NBUF):
            gather_wait(b)
            scatter(b, b)
            bn = (b + LOOK) % NBUF
            if b + LOOK >= NBUF:
                scatter_wait(bn)
            gather(b + LOOK, bn)

        def grp(g, c):
            for b in range(NBUF):
                j = g * NBUF + b
                gather_wait(b)
                scatter(j, b)
                bn = (b + LOOK) % NBUF
                scatter_wait(bn)
                gather(j + LOOK, bn)
            return c

        lax.fori_loop(1, n_groups - 1, grp, 0)

        for b in range(NBUF):
            j = (n_groups - 1) * NBUF + b
            gather_wait(b)
            scatter(j, b)
            if j + LOOK < n_chunks:
                bn = (b + LOOK) % NBUF
                scatter_wait(bn)
                gather(j + LOOK, bn)

        for b in range(NBUF):
            scatter_wait(b)

    return body(ids3, table)


def kernel(input_ids, table):
    B, T = input_ids.shape
    D = table.shape[1]
    ids = input_ids.reshape(-1).astype(jnp.int32)
    info = plsc.get_sparse_core_info()
    nw = info.num_cores * info.num_subcores
    n_chunks = ids.shape[0] // (nw * CHUNK)
    out = _gather_rows(ids.reshape(nw, n_chunks, CHUNK), table)
    return out.reshape(B, T, D)


# speedup vs baseline: 1.0015x; 1.0015x over previous
import functools

import jax
import jax.numpy as jnp
from jax import lax
from jax.experimental import pallas as pl
from jax.experimental.pallas import tpu as pltpu
from jax.experimental.pallas import tpu_sc as plsc

CHUNK = 64
NBUF = 8
LOOK = 5


def _gather_rows(ids3, table):
    nw, n_chunks, _ = ids3.shape
    D = table.shape[1]
    N = nw * n_chunks * CHUNK
    rows_per_w = n_chunks * CHUNK
    info = plsc.get_sparse_core_info()
    nc = info.num_cores
    n_groups = n_chunks // NBUF

    mesh = plsc.VectorSubcoreMesh(core_axis_name="c", subcore_axis_name="s")

    @functools.partial(
        pl.kernel,
        mesh=mesh,
        out_type=jax.ShapeDtypeStruct((N, D), jnp.float32),
        scratch_types=[
            pltpu.VMEM((n_chunks, CHUNK), jnp.int32),
            *[pltpu.VMEM((CHUNK, D), jnp.float32) for _ in range(NBUF)],
            pltpu.SemaphoreType.DMA((NBUF,)),
            pltpu.SemaphoreType.DMA((NBUF,)),
        ],
    )
    def body(ids_hbm, table_hbm, out_hbm, idx_v, *rest):
        rows = rest[:NBUF]
        sem_g = rest[NBUF]
        sem_o = rest[NBUF + 1]
        wid = lax.axis_index("s") * nc + lax.axis_index("c")
        base_w = wid * rows_per_w

        pltpu.sync_copy(ids_hbm.at[wid], idx_v)

        def gather(j, b):
            pltpu.async_copy(table_hbm.at[idx_v.at[j]], rows[b], sem_g.at[b])

        def gather_wait(b):
            pltpu.make_async_copy(
                table_hbm.at[idx_v.at[0]], rows[b], sem_g.at[b]).wait()

        def scatter(j, b):
            pltpu.async_copy(
                rows[b], out_hbm.at[pl.ds(base_w + j * CHUNK, CHUNK)],
                sem_o.at[b])

        def scatter_wait(b):
            pltpu.make_async_copy(
                rows[b], out_hbm.at[pl.ds(base_w, CHUNK)], sem_o.at[b]).wait()

        for b in range(LOOK):
            gather(b, b)

        for b in range(NBUF):
            gather_wait(b)
            scatter(b, b)
            bn = (b + LOOK) % NBUF
            if b + LOOK >= NBUF:
                scatter_wait(bn)
            gather(b + LOOK, bn)

        def grp(g, c):
            for b in range(NBUF):
                j = g * NBUF + b
                gather_wait(b)
                scatter(j, b)
                bn = (b + LOOK) % NBUF
                scatter_wait(bn)
                gather(j + LOOK, bn)
            return c

        lax.fori_loop(1, n_groups - 1, grp, 0)

        for b in range(NBUF):
            j = (n_groups - 1) * NBUF + b
            gather_wait(b)
            scatter(j, b)
            if j + LOOK < n_chunks:
                bn = (b + LOOK) % NBUF
                scatter_wait(bn)
                gather(j + LOOK, bn)

        for b in range(NBUF):
            scatter_wait(b)

    return body(ids3, table)


def kernel(input_ids, table):
    B, T = input_ids.shape
    D = table.shape[1]
    ids = input_ids.reshape(-1).astype(jnp.int32)
    info = plsc.get_sparse_core_info()
    nw = info.num_cores * info.num_subcores
    n_chunks = ids.shape[0] // (nw * CHUNK)
    out = _gather_rows(ids.reshape(nw, n_chunks, CHUNK), table)
    return out.reshape(B, T, D)


# speedup vs baseline: 1.0052x; 1.0037x over previous
import functools

import jax
import jax.numpy as jnp
from jax import lax
from jax.experimental import pallas as pl
from jax.experimental.pallas import tpu as pltpu
from jax.experimental.pallas import tpu_sc as plsc

CHUNK = 128
NBUF = 5
LOOK = 3


def _gather_rows(ids3, table):
    nw, n_chunks, _ = ids3.shape
    D = table.shape[1]
    N = nw * n_chunks * CHUNK
    rows_per_w = n_chunks * CHUNK
    info = plsc.get_sparse_core_info()
    nc = info.num_cores
    n_groups = n_chunks // NBUF

    mesh = plsc.VectorSubcoreMesh(core_axis_name="c", subcore_axis_name="s")

    @functools.partial(
        pl.kernel,
        mesh=mesh,
        out_type=jax.ShapeDtypeStruct((N, D), jnp.float32),
        scratch_types=[
            pltpu.VMEM((n_chunks, CHUNK), jnp.int32),
            *[pltpu.VMEM((CHUNK, D), jnp.float32) for _ in range(NBUF)],
            pltpu.SemaphoreType.DMA((NBUF,)),
            pltpu.SemaphoreType.DMA((NBUF,)),
        ],
    )
    def body(ids_hbm, table_hbm, out_hbm, idx_v, *rest):
        rows = rest[:NBUF]
        sem_g = rest[NBUF]
        sem_o = rest[NBUF + 1]
        wid = lax.axis_index("s") * nc + lax.axis_index("c")
        base_w = wid * rows_per_w

        pltpu.sync_copy(ids_hbm.at[wid], idx_v)

        def gather(j, b):
            pltpu.async_copy(table_hbm.at[idx_v.at[j]], rows[b], sem_g.at[b])

        def gather_wait(b):
            pltpu.make_async_copy(
                table_hbm.at[idx_v.at[0]], rows[b], sem_g.at[b]).wait()

        def scatter(j, b):
            pltpu.async_copy(
                rows[b], out_hbm.at[pl.ds(base_w + j * CHUNK, CHUNK)],
                sem_o.at[b])

        def scatter_wait(b):
            pltpu.make_async_copy(
                rows[b], out_hbm.at[pl.ds(base_w, CHUNK)], sem_o.at[b]).wait()

        for b in range(LOOK):
            gather(b, b)

        for b in range(NBUF):
            gather_wait(b)
            scatter(b, b)
            bn = (b + LOOK) % NBUF
            if b + LOOK >= NBUF:
                scatter_wait(bn)
            gather(b + LOOK, bn)

        def grp(g, c):
            for b in range(NBUF):
                j = g * NBUF + b
                gather_wait(b)
                scatter(j, b)
                bn = (b + LOOK) % NBUF
                scatter_wait(bn)
                gather(j + LOOK, bn)
            return c

        lax.fori_loop(1, n_groups - 1, grp, 0)

        for b in range(NBUF):
            j = (n_groups - 1) * NBUF + b
            gather_wait(b)
            scatter(j, b)
            if j + LOOK < n_chunks:
                bn = (b + LOOK) % NBUF
                scatter_wait(bn)
                gather(j + LOOK, bn)

        for b in range(NBUF):
            scatter_wait(b)

    return body(ids3, table)


def kernel(input_ids, table):
    B, T = input_ids.shape
    D = table.shape[1]
    ids = input_ids.reshape(-1).astype(jnp.int32)
    info = plsc.get_sparse_core_info()
    nw = info.num_cores * info.num_subcores
    n_chunks = ids.shape[0] // (nw * CHUNK)
    out = _gather_rows(ids.reshape(nw, n_chunks, CHUNK), table)
    return out.reshape(B, T, D)
